# initial kernel scaffold (unmeasured)
import jax
import jax.numpy as jnp
from jax import lax
from jax.experimental import pallas as pl
from jax.experimental.pallas import tpu as pltpu

N_DEV = 4


def kernel(x, w_mat):
    m_per, k = x.shape
    k2, n_per = w_mat.shape
    assert k == k2

    def body(x_ref, w_ref, out_ref, comm_ref, send_sems, recv_sems):
        my_pos = lax.axis_index("i")
        left = (my_pos - 1) % N_DEV
        right = (my_pos + 1) % N_DEV

        barrier_sem = pltpu.get_barrier_semaphore()
        for nbr in [left, right]:
            pl.semaphore_signal(
                barrier_sem, inc=1,
                device_id=(nbr,), device_id_type=pl.DeviceIdType.MESH,
            )
        pl.semaphore_wait(barrier_sem, 2)

        comm_ref[0, :, :] = x_ref[:, :]
        local = jnp.dot(x_ref[:, :], w_ref[:, :],
                        preferred_element_type=jnp.float32)
        out_ref[pl.ds(my_pos * m_per, m_per), :] = jnp.maximum(local, 0.0)

        for h in range(N_DEV - 1):
            send_slot = h % 2
            recv_slot = (h + 1) % 2
            rdma = pltpu.make_async_remote_copy(
                src_ref=comm_ref.at[send_slot],
                dst_ref=comm_ref.at[recv_slot],
                send_sem=send_sems.at[send_slot],
                recv_sem=recv_sems.at[recv_slot],
                device_id=(right,),
                device_id_type=pl.DeviceIdType.MESH,
            )
            rdma.start()
            rdma.wait()

            origin = (my_pos - h - 1) % N_DEV
            blk = jnp.dot(comm_ref[recv_slot, :, :], w_ref[:, :],
                          preferred_element_type=jnp.float32)
            out_ref[pl.ds(origin * m_per, m_per), :] = jnp.maximum(blk, 0.0)

    return pl.pallas_call(
        body,
        out_shape=jax.ShapeDtypeStruct((N_DEV * m_per, n_per), jnp.float32),
        in_specs=[
            pl.BlockSpec(memory_space=pltpu.VMEM),
            pl.BlockSpec(memory_space=pltpu.VMEM),
        ],
        out_specs=pl.BlockSpec(memory_space=pltpu.VMEM),
        scratch_shapes=[
            pltpu.VMEM((2, m_per, k), jnp.float32),
            pltpu.SemaphoreType.DMA((2,)),
            pltpu.SemaphoreType.DMA((2,)),
        ],
        compiler_params=pltpu.CompilerParams(collective_id=0),
    )(x, w_mat)


# baseline (device time: 344803 ns/iter reference)
import jax
import jax.numpy as jnp
from jax import lax
from jax.experimental import pallas as pl
from jax.experimental.pallas import tpu as pltpu

N_DEV = 4
NB = 512


def kernel(x, w_mat):
    m_per, k = x.shape
    k2, n_per = w_mat.shape
    assert k == k2
    half = m_per // 2
    n_tiles = n_per // NB

    def body(x_ref, w_ref, out_ref,
             cw_ref, ccw_ref, w_vmem, y_vmem,
             cw_send, cw_recv, ccw_send, ccw_recv,
             w_sems, y_sems, stage_sems):
        my_pos = lax.axis_index("i")
        left = (my_pos - 1) % N_DEV
        right = (my_pos + 1) % N_DEV

        barrier_sem = pltpu.get_barrier_semaphore()
        for nbr in [left, right]:
            pl.semaphore_signal(
                barrier_sem, inc=1,
                device_id=(nbr,), device_id_type=pl.DeviceIdType.MESH,
            )
        pl.semaphore_wait(barrier_sem, 2)

        st_cw = pltpu.make_async_copy(
            x_ref.at[pl.ds(0, half), :], cw_ref.at[0], stage_sems.at[0])
        st_ccw = pltpu.make_async_copy(
            x_ref.at[pl.ds(half, half), :], ccw_ref.at[0], stage_sems.at[1])
        st_cw.start()
        st_ccw.start()
        st_cw.wait()
        st_ccw.wait()

        def compute_chunk(c):
            slot = c % 2
            o_cw = (my_pos - c) % N_DEV
            o_ccw = (my_pos + c) % N_DEV
            for j in range(n_tiles):
                wslot = j % 2
                wcp = pltpu.make_async_copy(
                    w_ref.at[:, pl.ds(j * NB, NB)],
                    w_vmem.at[wslot], w_sems.at[wslot])
                wcp.start()
                wcp.wait()
                y_cw = jnp.dot(cw_ref[slot], w_vmem[wslot],
                               preferred_element_type=jnp.float32)
                y_vmem[0] = jnp.maximum(y_cw, 0.0)
                ocp0 = pltpu.make_async_copy(
                    y_vmem.at[0],
                    out_ref.at[pl.ds(o_cw * m_per, half),
                               pl.ds(j * NB, NB)],
                    y_sems.at[0])
                ocp0.start()
                y_ccw = jnp.dot(ccw_ref[slot], w_vmem[wslot],
                                preferred_element_type=jnp.float32)
                y_vmem[1] = jnp.maximum(y_ccw, 0.0)
                ocp1 = pltpu.make_async_copy(
                    y_vmem.at[1],
                    out_ref.at[pl.ds(o_ccw * m_per + half, half),
                               pl.ds(j * NB, NB)],
                    y_sems.at[1])
                ocp1.start()
                ocp0.wait()
                ocp1.wait()

        for h in range(1, N_DEV):
            s, d = (h - 1) % 2, h % 2
            rdma_cw = pltpu.make_async_remote_copy(
                src_ref=cw_ref.at[s], dst_ref=cw_ref.at[d],
                send_sem=cw_send.at[s], recv_sem=cw_recv.at[d],
                device_id=(right,), device_id_type=pl.DeviceIdType.MESH,
            )
            rdma_ccw = pltpu.make_async_remote_copy(
                src_ref=ccw_ref.at[s], dst_ref=ccw_ref.at[d],
                send_sem=ccw_send.at[s], recv_sem=ccw_recv.at[d],
                device_id=(left,), device_id_type=pl.DeviceIdType.MESH,
            )
            rdma_cw.start()
            rdma_ccw.start()
            compute_chunk(h - 1)
            rdma_cw.wait()
            rdma_ccw.wait()
        compute_chunk(N_DEV - 1)

    return pl.pallas_call(
        body,
        out_shape=jax.ShapeDtypeStruct((N_DEV * m_per, n_per), jnp.float32),
        in_specs=[
            pl.BlockSpec(memory_space=pl.ANY),
            pl.BlockSpec(memory_space=pl.ANY),
        ],
        out_specs=pl.BlockSpec(memory_space=pl.ANY),
        scratch_shapes=[
            pltpu.VMEM((2, half, k), jnp.float32),
            pltpu.VMEM((2, half, k), jnp.float32),
            pltpu.VMEM((2, k, NB), jnp.float32),
            pltpu.VMEM((2, half, NB), jnp.float32),
            pltpu.SemaphoreType.DMA((2,)),
            pltpu.SemaphoreType.DMA((2,)),
            pltpu.SemaphoreType.DMA((2,)),
            pltpu.SemaphoreType.DMA((2,)),
            pltpu.SemaphoreType.DMA((2,)),
            pltpu.SemaphoreType.DMA((2,)),
            pltpu.SemaphoreType.DMA((2,)),
        ],
        compiler_params=pltpu.CompilerParams(
            collective_id=0, vmem_limit_bytes=64 * 1024 * 1024),
    )(x, w_mat)


# device time: 319843 ns/iter; 1.0780x vs baseline; 1.0780x over previous
import jax
import jax.numpy as jnp
from jax import lax
from jax.experimental import pallas as pl
from jax.experimental.pallas import tpu as pltpu

N_DEV = 4
NB = 512


def kernel(x, w_mat):
    m_per, k = x.shape
    k2, n_per = w_mat.shape
    assert k == k2
    half = m_per // 2
    quart = half // 2
    n_tiles = n_per // NB

    def body(x_ref, w_ref, out_ref,
             cw_ref, ccw_ref, w_vmem, y_vmem,
             cw_send, cw_recv, ccw_send, ccw_recv,
             w_sems, y_sems, stage_sems):
        my_pos = lax.axis_index("i")
        left = (my_pos - 1) % N_DEV
        right = (my_pos + 1) % N_DEV

        barrier_sem = pltpu.get_barrier_semaphore()
        for nbr in [left, right]:
            pl.semaphore_signal(
                barrier_sem, inc=1,
                device_id=(nbr,), device_id_type=pl.DeviceIdType.MESH,
            )
        pl.semaphore_wait(barrier_sem, 2)

        pending = {}

        def drain(slot):
            cp = pending.pop(slot, None)
            if cp is not None:
                cp.wait()

        def compute_rows(slot, c, r0, nr):
            o_cw = (my_pos - c) % N_DEV
            o_ccw = (my_pos + c) % N_DEV
            wcps = [
                pltpu.make_async_copy(
                    w_ref.at[:, pl.ds(j * NB, NB)],
                    w_vmem.at[j % 2], w_sems.at[j % 2])
                for j in range(n_tiles)
            ]
            wcps[0].start()
            for j in range(n_tiles):
                ws = j % 2
                wcps[j].wait()
                if j + 1 < n_tiles:
                    wcps[j + 1].start()
                y_cw = jnp.dot(cw_ref[slot, pl.ds(r0, nr), :], w_vmem[ws],
                               preferred_element_type=jnp.float32)
                drain(0)
                y_vmem[0, pl.ds(0, nr), :] = jnp.maximum(y_cw, 0.0)
                y_cw_cp = pltpu.make_async_copy(
                    y_vmem.at[0, pl.ds(0, nr), :],
                    out_ref.at[pl.ds(o_cw * m_per + r0, nr),
                               pl.ds(j * NB, NB)],
                    y_sems.at[0])
                y_cw_cp.start()
                pending[0] = y_cw_cp
                y_ccw = jnp.dot(ccw_ref[slot, pl.ds(r0, nr), :], w_vmem[ws],
                                preferred_element_type=jnp.float32)
                drain(1)
                y_vmem[1, pl.ds(0, nr), :] = jnp.maximum(y_ccw, 0.0)
                y_ccw_cp = pltpu.make_async_copy(
                    y_vmem.at[1, pl.ds(0, nr), :],
                    out_ref.at[pl.ds(o_ccw * m_per + half + r0, nr),
                               pl.ds(j * NB, NB)],
                    y_sems.at[1])
                y_ccw_cp.start()
                pending[1] = y_ccw_cp


        rdma1_cw = pltpu.make_async_remote_copy(
            src_ref=x_ref.at[pl.ds(0, half), :], dst_ref=cw_ref.at[1],
            send_sem=cw_send.at[0], recv_sem=cw_recv.at[0],
            device_id=(right,), device_id_type=pl.DeviceIdType.MESH)
        rdma1_ccw = pltpu.make_async_remote_copy(
            src_ref=x_ref.at[pl.ds(half, half), :], dst_ref=ccw_ref.at[1],
            send_sem=ccw_send.at[0], recv_sem=ccw_recv.at[0],
            device_id=(left,), device_id_type=pl.DeviceIdType.MESH)
        rdma1_cw.start()
        rdma1_ccw.start()

        st_cw = pltpu.make_async_copy(
            x_ref.at[pl.ds(0, half), :], cw_ref.at[0], stage_sems.at[0])
        st_ccw = pltpu.make_async_copy(
            x_ref.at[pl.ds(half, half), :], ccw_ref.at[0], stage_sems.at[1])
        st_cw.start()
        st_ccw.start()
        st_cw.wait()
        st_ccw.wait()

        compute_rows(0, 0, 0, half)
        rdma1_cw.wait()
        rdma1_ccw.wait()

        rdma2_cw = pltpu.make_async_remote_copy(
            src_ref=cw_ref.at[1], dst_ref=cw_ref.at[0],
            send_sem=cw_send.at[1], recv_sem=cw_recv.at[1],
            device_id=(right,), device_id_type=pl.DeviceIdType.MESH)
        rdma2_ccw = pltpu.make_async_remote_copy(
            src_ref=ccw_ref.at[1], dst_ref=ccw_ref.at[0],
            send_sem=ccw_send.at[1], recv_sem=ccw_recv.at[1],
            device_id=(left,), device_id_type=pl.DeviceIdType.MESH)
        rdma2_cw.start()
        rdma2_ccw.start()
        compute_rows(1, 1, 0, half)
        rdma2_cw.wait()
        rdma2_ccw.wait()

        rdma3 = []
        for q in range(2):
            rdma3.append(pltpu.make_async_remote_copy(
                src_ref=cw_ref.at[0, pl.ds(q * quart, quart), :],
                dst_ref=cw_ref.at[1, pl.ds(q * quart, quart), :],
                send_sem=cw_send.at[2 + q], recv_sem=cw_recv.at[2 + q],
                device_id=(right,), device_id_type=pl.DeviceIdType.MESH))
            rdma3.append(pltpu.make_async_remote_copy(
                src_ref=ccw_ref.at[0, pl.ds(q * quart, quart), :],
                dst_ref=ccw_ref.at[1, pl.ds(q * quart, quart), :],
                send_sem=ccw_send.at[2 + q], recv_sem=ccw_recv.at[2 + q],
                device_id=(left,), device_id_type=pl.DeviceIdType.MESH))
        for r in rdma3:
            r.start()
        compute_rows(0, 2, 0, half)
        rdma3[0].wait_recv()
        rdma3[1].wait_recv()
        compute_rows(1, 3, 0, quart)
        rdma3[2].wait_recv()
        rdma3[3].wait_recv()
        for r in rdma3:
            r.wait_send()
        compute_rows(1, 3, quart, quart)

        drain(0)
        drain(1)

    return pl.pallas_call(
        body,
        out_shape=jax.ShapeDtypeStruct((N_DEV * m_per, n_per), jnp.float32),
        in_specs=[
            pl.BlockSpec(memory_space=pl.ANY),
            pl.BlockSpec(memory_space=pl.ANY),
        ],
        out_specs=pl.BlockSpec(memory_space=pl.ANY),
        scratch_shapes=[
            pltpu.VMEM((2, half, k), jnp.float32),
            pltpu.VMEM((2, half, k), jnp.float32),
            pltpu.VMEM((2, k, NB), jnp.float32),
            pltpu.VMEM((2, half, NB), jnp.float32),
            pltpu.SemaphoreType.DMA((4,)),
            pltpu.SemaphoreType.DMA((4,)),
            pltpu.SemaphoreType.DMA((4,)),
            pltpu.SemaphoreType.DMA((4,)),
            pltpu.SemaphoreType.DMA((2,)),
            pltpu.SemaphoreType.DMA((2,)),
            pltpu.SemaphoreType.DMA((2,)),
        ],
        compiler_params=pltpu.CompilerParams(
            collective_id=0, vmem_limit_bytes=64 * 1024 * 1024),
    )(x, w_mat)


# device time: 319128 ns/iter; 1.0805x vs baseline; 1.0022x over previous
import jax
import jax.numpy as jnp
from jax import lax
from jax.experimental import pallas as pl
from jax.experimental.pallas import tpu as pltpu

N_DEV = 4
CACHE_COLS = 1024
NBC = 512
NBS = 256


def kernel(x, w_mat):
    m_per, k = x.shape
    k2, n_per = w_mat.shape
    assert k == k2
    half = m_per // 2
    quart = half // 2
    n_cached = CACHE_COLS // NBC
    n_stream = (n_per - CACHE_COLS) // NBS

    def body(x_ref, w_ref, out_ref,
             cw_ref, ccw_ref, w_cache, w_stream, y_vmem,
             cw_send, cw_recv, ccw_send, ccw_recv,
             ws_sems, y_sems, stage_sems, wc_sem):
        my_pos = lax.axis_index("i")
        left = (my_pos - 1) % N_DEV
        right = (my_pos + 1) % N_DEV

        barrier_sem = pltpu.get_barrier_semaphore()
        for nbr in [left, right]:
            pl.semaphore_signal(
                barrier_sem, inc=1,
                device_id=(nbr,), device_id_type=pl.DeviceIdType.MESH,
            )
        pl.semaphore_wait(barrier_sem, 2)

        pending = {}

        def drain(slot):
            cp = pending.pop(slot, None)
            if cp is not None:
                cp.wait()

        def compute_rows(slot, c, r0, nr):
            o_cw = (my_pos - c) % N_DEV
            o_ccw = (my_pos + c) % N_DEV
            bases = (o_cw * m_per + r0, o_ccw * m_per + half + r0)
            srcs = (cw_ref, ccw_ref)

            def emit(d, vals, col0, width):
                drain(d)
                y_vmem[d, pl.ds(0, nr), pl.ds(0, width)] = \
                    jnp.maximum(vals, 0.0)
                cp = pltpu.make_async_copy(
                    y_vmem.at[d, pl.ds(0, nr), pl.ds(0, width)],
                    out_ref.at[pl.ds(bases[d], nr), pl.ds(col0, width)],
                    y_sems.at[d])
                cp.start()
                pending[d] = cp

            scps = [
                pltpu.make_async_copy(
                    w_ref.at[:, pl.ds(CACHE_COLS + s * NBS, NBS)],
                    w_stream.at[s % 2], ws_sems.at[s % 2])
                for s in range(n_stream)
            ]
            scps[0].start()
            scps[1].start()
            for t in range(n_cached):
                for d in range(2):
                    y = jnp.dot(srcs[d][slot, pl.ds(r0, nr), :],
                                w_cache[:, pl.ds(t * NBC, NBC)],
                                preferred_element_type=jnp.float32)
                    emit(d, y, t * NBC, NBC)
            for s in range(n_stream):
                scps[s].wait()
                for d in range(2):
                    y = jnp.dot(srcs[d][slot, pl.ds(r0, nr), :],
                                w_stream[s % 2],
                                preferred_element_type=jnp.float32)
                    emit(d, y, CACHE_COLS + s * NBS, NBS)
                if s + 2 < n_stream:
                    scps[s + 2].start()

        rdma1_cw = pltpu.make_async_remote_copy(
            src_ref=x_ref.at[pl.ds(0, half), :], dst_ref=cw_ref.at[1],
            send_sem=cw_send.at[0], recv_sem=cw_recv.at[0],
            device_id=(right,), device_id_type=pl.DeviceIdType.MESH)
        rdma1_ccw = pltpu.make_async_remote_copy(
            src_ref=x_ref.at[pl.ds(half, half), :], dst_ref=ccw_ref.at[1],
            send_sem=ccw_send.at[0], recv_sem=ccw_recv.at[0],
            device_id=(left,), device_id_type=pl.DeviceIdType.MESH)
        rdma1_cw.start()
        rdma1_ccw.start()

        wc_cp = pltpu.make_async_copy(
            w_ref.at[:, pl.ds(0, CACHE_COLS)], w_cache, wc_sem)
        wc_cp.start()
        st_cw = pltpu.make_async_copy(
            x_ref.at[pl.ds(0, half), :], cw_ref.at[0], stage_sems.at[0])
        st_ccw = pltpu.make_async_copy(
            x_ref.at[pl.ds(half, half), :], ccw_ref.at[0], stage_sems.at[1])
        st_cw.start()
        st_ccw.start()
        st_cw.wait()
        st_ccw.wait()
        wc_cp.wait()

        compute_rows(0, 0, 0, half)
        rdma1_cw.wait()
        rdma1_ccw.wait()

        rdma2_cw = pltpu.make_async_remote_copy(
            src_ref=cw_ref.at[1], dst_ref=cw_ref.at[0],
            send_sem=cw_send.at[1], recv_sem=cw_recv.at[1],
            device_id=(right,), device_id_type=pl.DeviceIdType.MESH)
        rdma2_ccw = pltpu.make_async_remote_copy(
            src_ref=ccw_ref.at[1], dst_ref=ccw_ref.at[0],
            send_sem=ccw_send.at[1], recv_sem=ccw_recv.at[1],
            device_id=(left,), device_id_type=pl.DeviceIdType.MESH)
        rdma2_cw.start()
        rdma2_ccw.start()
        compute_rows(1, 1, 0, half)
        rdma2_cw.wait()
        rdma2_ccw.wait()

        rdma3 = []
        for q in range(2):
            rdma3.append(pltpu.make_async_remote_copy(
                src_ref=cw_ref.at[0, pl.ds(q * quart, quart), :],
                dst_ref=cw_ref.at[1, pl.ds(q * quart, quart), :],
                send_sem=cw_send.at[2 + q], recv_sem=cw_recv.at[2 + q],
                device_id=(right,), device_id_type=pl.DeviceIdType.MESH))
            rdma3.append(pltpu.make_async_remote_copy(
                src_ref=ccw_ref.at[0, pl.ds(q * quart, quart), :],
                dst_ref=ccw_ref.at[1, pl.ds(q * quart, quart), :],
                send_sem=ccw_send.at[2 + q], recv_sem=ccw_recv.at[2 + q],
                device_id=(left,), device_id_type=pl.DeviceIdType.MESH))
        for r in rdma3:
            r.start()
        compute_rows(0, 2, 0, half)
        rdma3[0].wait_recv()
        rdma3[1].wait_recv()
        compute_rows(1, 3, 0, quart)
        rdma3[2].wait_recv()
        rdma3[3].wait_recv()
        for r in rdma3:
            r.wait_send()
        compute_rows(1, 3, quart, quart)

        drain(0)
        drain(1)

    return pl.pallas_call(
        body,
        out_shape=jax.ShapeDtypeStruct((N_DEV * m_per, n_per), jnp.float32),
        in_specs=[
            pl.BlockSpec(memory_space=pl.ANY),
            pl.BlockSpec(memory_space=pl.ANY),
        ],
        out_specs=pl.BlockSpec(memory_space=pl.ANY),
        scratch_shapes=[
            pltpu.VMEM((2, half, k), jnp.float32),
            pltpu.VMEM((2, half, k), jnp.float32),
            pltpu.VMEM((k, CACHE_COLS), jnp.float32),
            pltpu.VMEM((2, k, NBS), jnp.float32),
            pltpu.VMEM((2, half, NBC), jnp.float32),
            pltpu.SemaphoreType.DMA((4,)),
            pltpu.SemaphoreType.DMA((4,)),
            pltpu.SemaphoreType.DMA((4,)),
            pltpu.SemaphoreType.DMA((4,)),
            pltpu.SemaphoreType.DMA((2,)),
            pltpu.SemaphoreType.DMA((2,)),
            pltpu.SemaphoreType.DMA((2,)),
            pltpu.SemaphoreType.DMA,
        ],
        compiler_params=pltpu.CompilerParams(
            collective_id=0, vmem_limit_bytes=64 * 1024 * 1024),
    )(x, w_mat)
